# baseline (speedup 1.0000x reference)
import functools

import jax
import jax.numpy as jnp
from jax import lax
from jax.experimental import pallas as pl
from jax.experimental.pallas import tpu as pltpu
from jax.experimental.pallas import tpu_sc as plsc

NC = 2
NS = 16
NW = NC * NS

D = 64
UNIT = 128


def _make_gather(B):
    assert B % (UNIT * NW) == 0
    units_total = B // UNIT
    units_per_w = units_total // NW

    mesh = plsc.VectorSubcoreMesh(core_axis_name="c", subcore_axis_name="s")

    @functools.partial(
        pl.kernel,
        out_type=jax.ShapeDtypeStruct((B, D), jnp.float32),
        mesh=mesh,
        scratch_types=[
            pltpu.VMEM((1, UNIT), jnp.int32),
            pltpu.VMEM((UNIT, D), jnp.float32),
            pltpu.SemaphoreType.DMA,
        ],
        compiler_params=pltpu.CompilerParams(use_tc_tiling_on_sc=False),
    )
    def gather_kernel(table_hbm, idx_hbm, out_hbm, idx_v, rows_v, sem):
        wid = lax.axis_index("s") * NC + lax.axis_index("c")
        base_unit = wid * units_per_w

        def body(j, carry):
            g = base_unit + j
            pltpu.sync_copy(idx_hbm.at[pl.ds(g, 1)], idx_v)
            pltpu.async_copy(table_hbm.at[idx_v.at[0]], rows_v, sem).wait()
            pltpu.sync_copy(rows_v, out_hbm.at[pl.ds(g * UNIT, UNIT)])
            return carry

        lax.fori_loop(0, units_per_w, body, 0)

    return gather_kernel


def kernel(inputs, embeddings):
    batch, seq = inputs.shape
    B = batch * seq
    idx2d = inputs.reshape(B // UNIT, UNIT).astype(jnp.int32)
    out = _make_gather(B)(embeddings, idx2d)
    return out.reshape(batch, seq, D)


# speedup vs baseline: 1.1981x; 1.1981x over previous
import functools

import jax
import jax.numpy as jnp
from jax import lax
from jax.experimental import pallas as pl
from jax.experimental.pallas import tpu as pltpu
from jax.experimental.pallas import tpu_sc as plsc

NC = 2
NS = 16
NW = NC * NS

D = 64
UNIT = 128
NBUF = 8
STAGGER = 4


def _make_gather(B):
    assert B % (UNIT * NW) == 0
    units_per_w = B // (UNIT * NW)
    assert units_per_w % NBUF == 0 and units_per_w > 2 * NBUF
    rots = units_per_w // NBUF

    mesh = plsc.VectorSubcoreMesh(core_axis_name="c", subcore_axis_name="s")

    @functools.partial(
        pl.kernel,
        out_type=jax.ShapeDtypeStruct((B, D), jnp.float32),
        mesh=mesh,
        scratch_types=[
            pltpu.VMEM((units_per_w, UNIT), jnp.int32),
            pltpu.VMEM((NBUF, UNIT, D), jnp.float32),
            pltpu.SemaphoreType.DMA((NBUF,)),
            pltpu.SemaphoreType.DMA((NBUF,)),
        ],
        compiler_params=pltpu.CompilerParams(use_tc_tiling_on_sc=False),
    )
    def gather_kernel(table_hbm, idx_hbm, out_hbm, idx_v, rows_v, gsem, ssem):
        wid = lax.axis_index("s") * NC + lax.axis_index("c")
        base_unit = wid * units_per_w

        pltpu.sync_copy(idx_hbm.at[pl.ds(base_unit, units_per_w)], idx_v)

        def fire_gather(u, slot):
            pltpu.async_copy(
                table_hbm.at[idx_v.at[u]], rows_v.at[slot], gsem.at[slot])

        def wait_gather(u, slot):
            pltpu.make_async_copy(
                table_hbm.at[idx_v.at[u]], rows_v.at[slot],
                gsem.at[slot]).wait()

        def out_slice(u):
            return out_hbm.at[pl.ds((base_unit + u) * UNIT, UNIT)]

        def fire_store(u, slot):
            pltpu.async_copy(rows_v.at[slot], out_slice(u), ssem.at[slot])

        def wait_store(u, slot):
            pltpu.make_async_copy(
                rows_v.at[slot], out_slice(u), ssem.at[slot]).wait()

        for b in range(NBUF):
            fire_gather(b, b)
            if b >= STAGGER:
                v = b - STAGGER
                wait_gather(v, v)
                fire_store(v, v)

        def body(r, carry):
            for b in range(NBUF):
                u = r * NBUF + b
                wait_store(u - NBUF, b)
                fire_gather(u, b)
                v = u - STAGGER
                vslot = (b - STAGGER) % NBUF
                wait_gather(v, vslot)
                fire_store(v, vslot)
            return carry

        lax.fori_loop(1, rots, body, 0)

        last = units_per_w - NBUF
        for b in range(NBUF - STAGGER, NBUF):
            v = last + b
            wait_gather(v, b)
            fire_store(v, b)
        for b in range(NBUF):
            wait_store(last + b, b)

    return gather_kernel


def kernel(inputs, embeddings):
    batch, seq = inputs.shape
    B = batch * seq
    idx2d = inputs.reshape(B // UNIT, UNIT).astype(jnp.int32)
    out = _make_gather(B)(embeddings, idx2d)
    return out.reshape(batch, seq, D)
